# Initial kernel scaffold; baseline (speedup 1.0000x reference)
import jax
import jax.numpy as jnp
from jax.experimental import pallas as pl


def kernel(x, edge_index, weight):
    raise NotImplementedError("write your pallas kernel here")


# speedup vs baseline: 54.6954x; 54.6954x over previous
import functools

import jax
import jax.numpy as jnp
from jax import lax
from jax.experimental import pallas as pl
from jax.experimental.pallas import tpu as pltpu
from jax.experimental.pallas import tpu_sc as plsc

N = 100000
F = 128
C = 16
E = 3200000
NC = 2
NS = 16
NW = NC * NS
SB = 128
NSUB = 8
EB = SB * NSUB
NB = 98
EPAD = NW * NB * EB
NP = 100096
RPS = NP // NS

_mesh = plsc.VectorSubcoreMesh(core_axis_name="c", subcore_axis_name="s")
_sc_params = pltpu.CompilerParams(use_tc_tiling_on_sc=False)


def _sc_propagate(g, rows4, cols4, zeros):
    @functools.partial(
        pl.kernel,
        out_type=jax.ShapeDtypeStruct((NC, NP, C), jnp.float32),
        mesh=_mesh,
        scratch_types=[
            pltpu.VMEM((NSUB, SB), jnp.int32),
            pltpu.VMEM((NSUB, SB), jnp.int32),
            pltpu.VMEM((EB, C), jnp.float32),
            pltpu.VMEM_SHARED((NP, C), jnp.float32),
            pltpu.SemaphoreType.DMA,
        ],
        compiler_params=_sc_params,
    )
    def k(g_hbm, row_hbm, col_hbm, z_hbm, out_hbm, ridx, cidx, buf, acc, sem):
        cid = lax.axis_index("c")
        sid = lax.axis_index("s")
        wid = cid * NS + sid
        pltpu.sync_copy(z_hbm.at[pl.ds(sid * RPS, RPS)],
                        acc.at[pl.ds(sid * RPS, RPS)])
        plsc.subcore_barrier()

        @pl.loop(0, NB)
        def _(b):
            pltpu.sync_copy(row_hbm.at[wid, b], ridx)
            pltpu.sync_copy(col_hbm.at[wid, b], cidx)
            descs = [
                pltpu.async_copy(g_hbm.at[ridx.at[j]],
                                 buf.at[pl.ds(j * SB, SB)], sem)
                for j in range(NSUB)
            ]
            for d in descs:
                d.wait()
            for j in range(NSUB):
                pltpu.sync_copy(buf.at[pl.ds(j * SB, SB)],
                                acc.at[cidx.at[j]], add=True)

        plsc.subcore_barrier()
        pltpu.sync_copy(acc.at[pl.ds(sid * RPS, RPS)],
                        out_hbm.at[cid, pl.ds(sid * RPS, RPS)])

    return k(g, rows4, cols4, zeros)


def _sc_degree(rows4, ones, zeros):
    @functools.partial(
        pl.kernel,
        out_type=jax.ShapeDtypeStruct((NC, NP, C), jnp.float32),
        mesh=_mesh,
        scratch_types=[
            pltpu.VMEM((NSUB, SB), jnp.int32),
            pltpu.VMEM((SB, C), jnp.float32),
            pltpu.VMEM_SHARED((NP, C), jnp.float32),
        ],
        compiler_params=_sc_params,
    )
    def k(row_hbm, ones_hbm, z_hbm, out_hbm, ridx, onesb, acc):
        cid = lax.axis_index("c")
        sid = lax.axis_index("s")
        wid = cid * NS + sid
        pltpu.sync_copy(ones_hbm, onesb)
        pltpu.sync_copy(z_hbm.at[pl.ds(sid * RPS, RPS)],
                        acc.at[pl.ds(sid * RPS, RPS)])
        plsc.subcore_barrier()

        @pl.loop(0, NB)
        def _(b):
            pltpu.sync_copy(row_hbm.at[wid, b], ridx)
            for j in range(NSUB):
                pltpu.sync_copy(onesb, acc.at[ridx.at[j]], add=True)

        plsc.subcore_barrier()
        pltpu.sync_copy(acc.at[pl.ds(sid * RPS, RPS)],
                        out_hbm.at[cid, pl.ds(sid * RPS, RPS)])

    return k(rows4, ones, zeros)


def _tc_matmul(x, w):
    BM = 2000

    def body(x_ref, w_ref, o_ref):
        o_ref[...] = jnp.dot(x_ref[...], w_ref[...],
                             preferred_element_type=jnp.float32)

    return pl.pallas_call(
        body,
        grid=(N // BM,),
        in_specs=[
            pl.BlockSpec((BM, F), lambda i: (i, 0)),
            pl.BlockSpec((F, C), lambda i: (0, 0)),
        ],
        out_specs=pl.BlockSpec((BM, C), lambda i: (i, 0)),
        out_shape=jax.ShapeDtypeStruct((N, C), jnp.float32),
    )(x, w)


_R = N * C // 128


def _tc_prep(dega, degb, h0f):
    def body(da, db, h, g0, di_o):
        deg = da[...] + db[...]
        di = jnp.where(deg > 0.0, lax.rsqrt(deg), 0.0)
        di_o[...] = di
        g0[...] = di * h[...]

    return pl.pallas_call(
        body,
        out_shape=[jax.ShapeDtypeStruct((_R, 128), jnp.float32)] * 2,
    )(dega, degb, h0f)


def _tc_combine(sa, sb, scale, square):
    def body(a, b, s, o):
        f = s[...]
        if square:
            f = f * f
        o[...] = f * (a[...] + b[...])

    return pl.pallas_call(
        body,
        out_shape=jax.ShapeDtypeStruct((_R, 128), jnp.float32),
    )(sa, sb, scale)


def _pad_nodes(gf):
    g = gf.reshape(N, C)
    return jnp.concatenate([g, jnp.zeros((NP - N, C), jnp.float32)], axis=0)


def kernel(x, edge_index, weight):
    row = edge_index[0].astype(jnp.int32)
    col = edge_index[1].astype(jnp.int32)
    padv = jnp.full((EPAD - E,), N, jnp.int32)
    rows4 = jnp.concatenate([row, padv]).reshape(NW, NB, NSUB, SB)
    cols4 = jnp.concatenate([col, padv]).reshape(NW, NB, NSUB, SB)
    zeros = jnp.zeros((NP, C), jnp.float32)
    ones = jnp.ones((SB, C), jnp.float32)

    deg_p = _sc_degree(rows4, ones, zeros)
    h0 = _tc_matmul(x, weight)

    dega = deg_p[0, :N].reshape(_R, 128)
    degb = deg_p[1, :N].reshape(_R, 128)
    g0f, dinvf = _tc_prep(dega, degb, h0.reshape(_R, 128))

    s1 = _sc_propagate(_pad_nodes(g0f), rows4, cols4, zeros)
    g1f = _tc_combine(s1[0, :N].reshape(_R, 128),
                      s1[1, :N].reshape(_R, 128), dinvf, square=True)

    s2 = _sc_propagate(_pad_nodes(g1f), rows4, cols4, zeros)
    h2f = _tc_combine(s2[0, :N].reshape(_R, 128),
                      s2[1, :N].reshape(_R, 128), dinvf, square=False)

    return h2f.reshape(N, C)


# speedup vs baseline: 66.2759x; 1.2117x over previous
import functools

import jax
import jax.numpy as jnp
from jax import lax
from jax.experimental import pallas as pl
from jax.experimental.pallas import tpu as pltpu
from jax.experimental.pallas import tpu_sc as plsc

N = 100000
F = 128
C = 16
E = 3200000
NC = 2
NS = 16
NW = NC * NS
SB = 128
NSUB = 8
EB = SB * NSUB
NB = 98
EPAD = NW * NB * EB
NP = 100096
RPS = NP // NS

_mesh = plsc.VectorSubcoreMesh(core_axis_name="c", subcore_axis_name="s")
_sc_params = pltpu.CompilerParams(use_tc_tiling_on_sc=False)
_f32 = jnp.float32


def _sc_propagate(g, rows4, cols4, zeros):
    @functools.partial(
        pl.kernel,
        out_type=[jax.ShapeDtypeStruct((NP, C), _f32)] * 2,
        mesh=_mesh,
        scratch_types=[
            pltpu.VMEM((NSUB, SB), jnp.int32),
            pltpu.VMEM((NSUB, SB), jnp.int32),
            pltpu.VMEM((EB, C), _f32),
            pltpu.VMEM_SHARED((NP, C), _f32),
            pltpu.SemaphoreType.DMA,
        ],
        compiler_params=_sc_params,
    )
    def k(g_hbm, row_hbm, col_hbm, z_hbm, oa_hbm, ob_hbm, ridx, cidx, buf,
          acc, sem):
        cid = lax.axis_index("c")
        sid = lax.axis_index("s")
        wid = cid * NS + sid
        pltpu.sync_copy(z_hbm.at[pl.ds(sid * RPS, RPS)],
                        acc.at[pl.ds(sid * RPS, RPS)])
        plsc.subcore_barrier()

        @pl.loop(0, NB)
        def _(b):
            pltpu.sync_copy(row_hbm.at[wid, b], ridx)
            pltpu.sync_copy(col_hbm.at[wid, b], cidx)
            descs = [
                pltpu.async_copy(g_hbm.at[ridx.at[j]],
                                 buf.at[pl.ds(j * SB, SB)], sem)
                for j in range(NSUB)
            ]
            for d in descs:
                d.wait()
            for j in range(NSUB):
                pltpu.sync_copy(buf.at[pl.ds(j * SB, SB)],
                                acc.at[cidx.at[j]], add=True)

        plsc.subcore_barrier()
        sl = pl.ds(sid * RPS, RPS)

        @pl.when(cid == 0)
        def _():
            pltpu.sync_copy(acc.at[sl], oa_hbm.at[sl])

        @pl.when(cid == 1)
        def _():
            pltpu.sync_copy(acc.at[sl], ob_hbm.at[sl])

    return k(g, rows4, cols4, zeros)


def _sc_degree(rows4, ones, zeros):
    @functools.partial(
        pl.kernel,
        out_type=[jax.ShapeDtypeStruct((NP, C), _f32)] * 2,
        mesh=_mesh,
        scratch_types=[
            pltpu.VMEM((NSUB, SB), jnp.int32),
            pltpu.VMEM((SB, C), _f32),
            pltpu.VMEM_SHARED((NP, C), _f32),
        ],
        compiler_params=_sc_params,
    )
    def k(row_hbm, ones_hbm, z_hbm, oa_hbm, ob_hbm, ridx, onesb, acc):
        cid = lax.axis_index("c")
        sid = lax.axis_index("s")
        wid = cid * NS + sid
        pltpu.sync_copy(ones_hbm, onesb)
        pltpu.sync_copy(z_hbm.at[pl.ds(sid * RPS, RPS)],
                        acc.at[pl.ds(sid * RPS, RPS)])
        plsc.subcore_barrier()

        @pl.loop(0, NB)
        def _(b):
            pltpu.sync_copy(row_hbm.at[wid, b], ridx)
            for j in range(NSUB):
                pltpu.sync_copy(onesb, acc.at[ridx.at[j]], add=True)

        plsc.subcore_barrier()
        sl = pl.ds(sid * RPS, RPS)

        @pl.when(cid == 0)
        def _():
            pltpu.sync_copy(acc.at[sl], oa_hbm.at[sl])

        @pl.when(cid == 1)
        def _():
            pltpu.sync_copy(acc.at[sl], ob_hbm.at[sl])

    return k(rows4, ones, zeros)


def _tc_matmul(x, w):
    BM = 2000

    def body(x_ref, w_ref, o_ref):
        o_ref[...] = jnp.dot(x_ref[...], w_ref[...],
                             preferred_element_type=_f32)

    return pl.pallas_call(
        body,
        grid=(N // BM,),
        in_specs=[
            pl.BlockSpec((BM, F), lambda i: (i, 0)),
            pl.BlockSpec((F, C), lambda i: (0, 0)),
        ],
        out_specs=pl.BlockSpec((BM, C), lambda i: (i, 0)),
        out_shape=jax.ShapeDtypeStruct((NP, C), _f32),
    )(x, w)


_BM = 6256


def _tc_prep(dega, degb, h0):
    def body(da, db, h, g0_o, di_o, di2_o):
        deg = da[...] + db[...]
        pos = deg > 0.0
        di = jnp.where(pos, lax.rsqrt(deg), 0.0)
        di_o[...] = di
        di2_o[...] = jnp.where(pos, 1.0 / deg, 0.0)
        g0_o[...] = di * h[...]

    return pl.pallas_call(
        body,
        grid=(NP // _BM,),
        in_specs=[pl.BlockSpec((_BM, C), lambda i: (i, 0))] * 3,
        out_specs=[pl.BlockSpec((_BM, C), lambda i: (i, 0))] * 3,
        out_shape=[jax.ShapeDtypeStruct((NP, C), _f32)] * 3,
    )(dega, degb, h0)


def _tc_combine(sa, sb, scale):
    def body(a, b, s, o):
        o[...] = s[...] * (a[...] + b[...])

    return pl.pallas_call(
        body,
        grid=(NP // _BM,),
        in_specs=[pl.BlockSpec((_BM, C), lambda i: (i, 0))] * 3,
        out_specs=pl.BlockSpec((_BM, C), lambda i: (i, 0)),
        out_shape=jax.ShapeDtypeStruct((NP, C), _f32),
    )(sa, sb, scale)


def _tc_finish(sa, sb, scale):
    BM = 2000

    def body(a, b, s, o):
        o[...] = s[...] * (a[...] + b[...])

    return pl.pallas_call(
        body,
        grid=(N // BM,),
        in_specs=[pl.BlockSpec((BM, C), lambda i: (i, 0))] * 3,
        out_specs=pl.BlockSpec((BM, C), lambda i: (i, 0)),
        out_shape=jax.ShapeDtypeStruct((N, C), _f32),
    )(sa, sb, scale)


def kernel(x, edge_index, weight):
    row = edge_index[0].astype(jnp.int32)
    col = edge_index[1].astype(jnp.int32)
    padv = jnp.full((EPAD - E,), N, jnp.int32)
    rows4 = jnp.concatenate([row, padv]).reshape(NW, NB, NSUB, SB)
    cols4 = jnp.concatenate([col, padv]).reshape(NW, NB, NSUB, SB)
    zeros = jnp.zeros((NP, C), _f32)
    ones = jnp.ones((SB, C), _f32)

    dega, degb = _sc_degree(rows4, ones, zeros)
    h0 = _tc_matmul(x, weight)
    g0, dinv, dinv2 = _tc_prep(dega, degb, h0)

    s1a, s1b = _sc_propagate(g0, rows4, cols4, zeros)
    g1 = _tc_combine(s1a, s1b, dinv2)

    s2a, s2b = _sc_propagate(g1, rows4, cols4, zeros)
    return _tc_finish(s2a, s2b, dinv)


# speedup vs baseline: 87.6995x; 1.3232x over previous
import functools

import jax
import jax.numpy as jnp
from jax import lax
from jax.experimental import pallas as pl
from jax.experimental.pallas import tpu as pltpu
from jax.experimental.pallas import tpu_sc as plsc

N = 100000
F = 128
C = 16
E = 3200000
NC = 2
NS = 16
NW = NC * NS
SB = 128
NSUB = 8
EB = SB * NSUB
NB = 98
EPAD = NW * NB * EB
NP = 100096
RPS = NP // NS

_mesh = plsc.VectorSubcoreMesh(core_axis_name="c", subcore_axis_name="s")
_sc_params = pltpu.CompilerParams(use_tc_tiling_on_sc=False)
_f32 = jnp.float32


def _sc_propagate(g, rows4, cols4, zeros):
    @functools.partial(
        pl.kernel,
        out_type=[jax.ShapeDtypeStruct((NP, C), _f32)] * 2,
        mesh=_mesh,
        scratch_types=[
            pltpu.VMEM((NSUB, SB), jnp.int32),
            pltpu.VMEM((NSUB, SB), jnp.int32),
            pltpu.VMEM((EB, C), _f32),
            pltpu.VMEM_SHARED((NP, C), _f32),
            pltpu.SemaphoreType.DMA,
        ],
        compiler_params=_sc_params,
    )
    def k(g_hbm, row_hbm, col_hbm, z_hbm, oa_hbm, ob_hbm, ridx, cidx, buf,
          acc, sem):
        cid = lax.axis_index("c")
        sid = lax.axis_index("s")
        wid = cid * NS + sid
        pltpu.sync_copy(z_hbm.at[pl.ds(sid * RPS, RPS)],
                        acc.at[pl.ds(sid * RPS, RPS)])
        plsc.subcore_barrier()

        @pl.loop(0, NB)
        def _(b):
            pltpu.sync_copy(row_hbm.at[wid, b], ridx)
            pltpu.sync_copy(col_hbm.at[wid, b], cidx)
            descs = [
                pltpu.async_copy(g_hbm.at[ridx.at[j]],
                                 buf.at[pl.ds(j * SB, SB)], sem)
                for j in range(NSUB)
            ]
            for d in descs:
                d.wait()
            for j in range(NSUB):
                pltpu.sync_copy(buf.at[pl.ds(j * SB, SB)],
                                acc.at[cidx.at[j]], add=True)

        plsc.subcore_barrier()
        sl = pl.ds(sid * RPS, RPS)

        @pl.when(cid == 0)
        def _():
            pltpu.sync_copy(acc.at[sl], oa_hbm.at[sl])

        @pl.when(cid == 1)
        def _():
            pltpu.sync_copy(acc.at[sl], ob_hbm.at[sl])

    return k(g, rows4, cols4, zeros)


def _sc_degree(rows4, ones, zeros):
    @functools.partial(
        pl.kernel,
        out_type=[jax.ShapeDtypeStruct((NP, C), _f32)] * 2,
        mesh=_mesh,
        scratch_types=[
            pltpu.VMEM((NSUB, SB), jnp.int32),
            pltpu.VMEM((SB, C), _f32),
            pltpu.VMEM_SHARED((NP, C), _f32),
        ],
        compiler_params=_sc_params,
    )
    def k(row_hbm, ones_hbm, z_hbm, oa_hbm, ob_hbm, ridx, onesb, acc):
        cid = lax.axis_index("c")
        sid = lax.axis_index("s")
        wid = cid * NS + sid
        pltpu.sync_copy(ones_hbm, onesb)
        pltpu.sync_copy(z_hbm.at[pl.ds(sid * RPS, RPS)],
                        acc.at[pl.ds(sid * RPS, RPS)])
        plsc.subcore_barrier()

        @pl.loop(0, NB)
        def _(b):
            pltpu.sync_copy(row_hbm.at[wid, b], ridx)
            for j in range(NSUB):
                pltpu.sync_copy(onesb, acc.at[ridx.at[j]], add=True)

        plsc.subcore_barrier()
        sl = pl.ds(sid * RPS, RPS)

        @pl.when(cid == 0)
        def _():
            pltpu.sync_copy(acc.at[sl], oa_hbm.at[sl])

        @pl.when(cid == 1)
        def _():
            pltpu.sync_copy(acc.at[sl], ob_hbm.at[sl])

    return k(rows4, ones, zeros)


def _tc_matmul(x, w):
    BM = 2000

    def body(x_ref, w_ref, o_ref):
        o_ref[...] = jnp.dot(x_ref[...], w_ref[...],
                             preferred_element_type=_f32)

    return pl.pallas_call(
        body,
        grid=(N // BM,),
        in_specs=[
            pl.BlockSpec((BM, F), lambda i: (i, 0)),
            pl.BlockSpec((F, C), lambda i: (0, 0)),
        ],
        out_specs=pl.BlockSpec((BM, C), lambda i: (i, 0)),
        out_shape=jax.ShapeDtypeStruct((NP, C), _f32),
    )(x, w)


_RF = NP * C // 128


def _flat(a):
    return a.reshape(_RF, 128)


def _tc_prep(dega, degb, h0):
    def body(da, db, h, g0_o, di_o, di2_o):
        deg = da[...] + db[...]
        pos = deg > 0.0
        di = jnp.where(pos, lax.rsqrt(deg), 0.0)
        di_o[...] = di
        di2_o[...] = jnp.where(pos, 1.0 / deg, 0.0)
        g0_o[...] = di * h[...]

    return pl.pallas_call(
        body,
        out_shape=[jax.ShapeDtypeStruct((_RF, 128), _f32)] * 3,
    )(_flat(dega), _flat(degb), _flat(h0))


def _tc_combine(sa, sb, scale):
    def body(a, b, s, o):
        o[...] = s[...] * (a[...] + b[...])

    return pl.pallas_call(
        body,
        out_shape=jax.ShapeDtypeStruct((_RF, 128), _f32),
    )(_flat(sa), _flat(sb), scale)


def kernel(x, edge_index, weight):
    row = edge_index[0].astype(jnp.int32)
    col = edge_index[1].astype(jnp.int32)
    padv = jnp.full((EPAD - E,), N, jnp.int32)
    rows4 = jnp.concatenate([row, padv]).reshape(NW, NB, NSUB, SB)
    cols4 = jnp.concatenate([col, padv]).reshape(NW, NB, NSUB, SB)
    zeros = jnp.zeros((NP, C), _f32)
    ones = jnp.ones((SB, C), _f32)

    dega, degb = _sc_degree(rows4, ones, zeros)
    h0 = _tc_matmul(x, weight)
    g0f, dinv, dinv2 = _tc_prep(dega, degb, h0)

    s1a, s1b = _sc_propagate(g0f.reshape(NP, C), rows4, cols4, zeros)
    g1f = _tc_combine(s1a, s1b, dinv2)

    s2a, s2b = _sc_propagate(g1f.reshape(NP, C), rows4, cols4, zeros)
    h2f = _tc_combine(s2a, s2b, dinv)
    return h2f.reshape(NP, C)[:N]


# speedup vs baseline: 107.7715x; 1.2289x over previous
import functools

import jax
import jax.numpy as jnp
from jax import lax
from jax.experimental import pallas as pl
from jax.experimental.pallas import tpu as pltpu
from jax.experimental.pallas import tpu_sc as plsc

N = 100000
F = 128
C = 16
E = 3200000
NC = 2
NS = 16
NW = NC * NS
SB = 128
NSUB = 4
EB = SB * NSUB
NB = 196
EPAD = NW * NB * EB
NP = 100096
RPS = NP // NS

_mesh = plsc.VectorSubcoreMesh(core_axis_name="c", subcore_axis_name="s")
_sc_params = pltpu.CompilerParams(use_tc_tiling_on_sc=False)
_f32 = jnp.float32


def _sc_propagate(g, rows4, cols4, zeros):
    @functools.partial(
        pl.kernel,
        out_type=[jax.ShapeDtypeStruct((NP, C), _f32)] * 2,
        mesh=_mesh,
        scratch_types=[
            pltpu.VMEM((2, NSUB, SB), jnp.int32),
            pltpu.VMEM((2, NSUB, SB), jnp.int32),
            pltpu.VMEM((2, EB, C), _f32),
            pltpu.VMEM_SHARED((NP, C), _f32),
            pltpu.SemaphoreType.DMA,
            pltpu.SemaphoreType.DMA,
            pltpu.SemaphoreType.DMA,
            pltpu.SemaphoreType.DMA,
        ],
        compiler_params=_sc_params,
    )
    def k(g_hbm, row_hbm, col_hbm, z_hbm, oa_hbm, ob_hbm, ridx, cidx, buf,
          acc, irs, ics, gs, ss):
        cid = lax.axis_index("c")
        sid = lax.axis_index("s")
        wid = cid * NS + sid
        pltpu.sync_copy(z_hbm.at[pl.ds(sid * RPS, RPS)],
                        acc.at[pl.ds(sid * RPS, RPS)])
        plsc.subcore_barrier()

        def issue_gathers(p, b):
            for j in range(NSUB):
                pltpu.async_copy(g_hbm.at[ridx.at[p, j]],
                                 buf.at[p, pl.ds(j * SB, SB)], gs)

        def issue_scatters(p):
            for j in range(NSUB):
                pltpu.async_copy(buf.at[p, pl.ds(j * SB, SB)],
                                 acc.at[cidx.at[p, j]], ss, add=True)

        def drain(sem):
            for j in range(NSUB):
                pltpu.make_async_copy(g_hbm.at[pl.ds(0, SB)],
                                      buf.at[0, pl.ds(j * SB, SB)],
                                      sem).wait()

        def step(b, p):
            q = 1 - p
            drain(gs)

            @pl.when(b + 2 < NB)
            def _():
                pltpu.async_copy(row_hbm.at[wid, b + 2], ridx.at[p], irs)

            @pl.when(b >= 1)
            def _():
                drain(ss)

            @pl.when(b + 1 < NB)
            def _():
                pltpu.async_copy(col_hbm.at[wid, b + 1], cidx.at[q], ics)
                pltpu.make_async_copy(row_hbm.at[wid, 0], ridx.at[0],
                                      irs).wait()
                issue_gathers(q, b + 1)

            @pl.when(b >= 1)
            def _():
                pltpu.make_async_copy(col_hbm.at[wid, 0], cidx.at[0],
                                      ics).wait()

            issue_scatters(p)

        pltpu.sync_copy(row_hbm.at[wid, 0], ridx.at[0])
        pltpu.sync_copy(col_hbm.at[wid, 0], cidx.at[0])
        issue_gathers(0, 0)
        pltpu.async_copy(row_hbm.at[wid, 1], ridx.at[1], irs)

        @pl.loop(0, NB // 2)
        def _(t):
            step(2 * t, 0)
            step(2 * t + 1, 1)

        drain(ss)
        plsc.subcore_barrier()
        sl = pl.ds(sid * RPS, RPS)

        @pl.when(cid == 0)
        def _():
            pltpu.sync_copy(acc.at[sl], oa_hbm.at[sl])

        @pl.when(cid == 1)
        def _():
            pltpu.sync_copy(acc.at[sl], ob_hbm.at[sl])

    return k(g, rows4, cols4, zeros)


def _sc_degree(rows4, ones, zeros):
    @functools.partial(
        pl.kernel,
        out_type=[jax.ShapeDtypeStruct((NP, C), _f32)] * 2,
        mesh=_mesh,
        scratch_types=[
            pltpu.VMEM((NSUB, SB), jnp.int32),
            pltpu.VMEM((SB, C), _f32),
            pltpu.VMEM_SHARED((NP, C), _f32),
        ],
        compiler_params=_sc_params,
    )
    def k(row_hbm, ones_hbm, z_hbm, oa_hbm, ob_hbm, ridx, onesb, acc):
        cid = lax.axis_index("c")
        sid = lax.axis_index("s")
        wid = cid * NS + sid
        pltpu.sync_copy(ones_hbm, onesb)
        pltpu.sync_copy(z_hbm.at[pl.ds(sid * RPS, RPS)],
                        acc.at[pl.ds(sid * RPS, RPS)])
        plsc.subcore_barrier()

        @pl.loop(0, NB)
        def _(b):
            pltpu.sync_copy(row_hbm.at[wid, b], ridx)
            for j in range(NSUB):
                pltpu.sync_copy(onesb, acc.at[ridx.at[j]], add=True)

        plsc.subcore_barrier()
        sl = pl.ds(sid * RPS, RPS)

        @pl.when(cid == 0)
        def _():
            pltpu.sync_copy(acc.at[sl], oa_hbm.at[sl])

        @pl.when(cid == 1)
        def _():
            pltpu.sync_copy(acc.at[sl], ob_hbm.at[sl])

    return k(rows4, ones, zeros)


def _tc_matmul(x, w):
    BM = 2000

    def body(x_ref, w_ref, o_ref):
        o_ref[...] = jnp.dot(x_ref[...], w_ref[...],
                             preferred_element_type=_f32)

    return pl.pallas_call(
        body,
        grid=(N // BM,),
        in_specs=[
            pl.BlockSpec((BM, F), lambda i: (i, 0)),
            pl.BlockSpec((F, C), lambda i: (0, 0)),
        ],
        out_specs=pl.BlockSpec((BM, C), lambda i: (i, 0)),
        out_shape=jax.ShapeDtypeStruct((NP, C), _f32),
    )(x, w)


_RF = NP * C // 128


def _flat(a):
    return a.reshape(_RF, 128)


def _tc_prep(dega, degb, h0):
    def body(da, db, h, g0_o, di_o, di2_o):
        deg = da[...] + db[...]
        pos = deg > 0.0
        di = jnp.where(pos, lax.rsqrt(deg), 0.0)
        di_o[...] = di
        di2_o[...] = jnp.where(pos, 1.0 / deg, 0.0)
        g0_o[...] = di * h[...]

    return pl.pallas_call(
        body,
        out_shape=[jax.ShapeDtypeStruct((_RF, 128), _f32)] * 3,
    )(_flat(dega), _flat(degb), _flat(h0))


def _tc_combine(sa, sb, scale):
    def body(a, b, s, o):
        o[...] = s[...] * (a[...] + b[...])

    return pl.pallas_call(
        body,
        out_shape=jax.ShapeDtypeStruct((_RF, 128), _f32),
    )(_flat(sa), _flat(sb), scale)


def kernel(x, edge_index, weight):
    row = edge_index[0].astype(jnp.int32)
    col = edge_index[1].astype(jnp.int32)
    padv = jnp.full((EPAD - E,), N, jnp.int32)
    rows4 = jnp.concatenate([row, padv]).reshape(NW, NB, NSUB, SB)
    cols4 = jnp.concatenate([col, padv]).reshape(NW, NB, NSUB, SB)
    zeros = jnp.zeros((NP, C), _f32)
    ones = jnp.ones((SB, C), _f32)

    dega, degb = _sc_degree(rows4, ones, zeros)
    h0 = _tc_matmul(x, weight)
    g0f, dinv, dinv2 = _tc_prep(dega, degb, h0)

    s1a, s1b = _sc_propagate(g0f.reshape(NP, C), rows4, cols4, zeros)
    g1f = _tc_combine(s1a, s1b, dinv2)

    s2a, s2b = _sc_propagate(g1f.reshape(NP, C), rows4, cols4, zeros)
    h2f = _tc_combine(s2a, s2b, dinv)
    return h2f.reshape(NP, C)[:N]


# speedup vs baseline: 119.3663x; 1.1076x over previous
import functools

import jax
import jax.numpy as jnp
from jax import lax
from jax.experimental import pallas as pl
from jax.experimental.pallas import tpu as pltpu
from jax.experimental.pallas import tpu_sc as plsc

N = 100000
F = 128
C = 16
E = 3200000
NC = 2
NS = 16
NW = NC * NS
SB = 128
NSUB = 4
EB = SB * NSUB
NB = 196
EPAD = NW * NB * EB
NP = 100096
RPS = NP // NS

_mesh = plsc.VectorSubcoreMesh(core_axis_name="c", subcore_axis_name="s")
_sc_params = pltpu.CompilerParams(use_tc_tiling_on_sc=False)
_f32 = jnp.float32


def _sc_propagate(g, rows4, cols4, zeros):
    @functools.partial(
        pl.kernel,
        out_type=[jax.ShapeDtypeStruct((NP, C), _f32)] * 2,
        mesh=_mesh,
        scratch_types=[
            pltpu.VMEM((2, NSUB, SB), jnp.int32),
            pltpu.VMEM((2, NSUB, SB), jnp.int32),
            pltpu.VMEM((2, EB, C), _f32),
            pltpu.VMEM_SHARED((NP, C), _f32),
            pltpu.SemaphoreType.DMA,
            pltpu.SemaphoreType.DMA,
            pltpu.SemaphoreType.DMA,
            pltpu.SemaphoreType.DMA,
        ],
        compiler_params=_sc_params,
    )
    def k(g_hbm, row_hbm, col_hbm, z_hbm, oa_hbm, ob_hbm, ridx, cidx, buf,
          acc, irs, ics, gs, ss):
        cid = lax.axis_index("c")
        sid = lax.axis_index("s")
        wid = cid * NS + sid
        pltpu.sync_copy(z_hbm.at[pl.ds(sid * RPS, RPS)],
                        acc.at[pl.ds(sid * RPS, RPS)])
        plsc.subcore_barrier()

        def issue_gathers(p, b):
            for j in range(NSUB):
                pltpu.async_copy(g_hbm.at[ridx.at[p, j]],
                                 buf.at[p, pl.ds(j * SB, SB)], gs)

        def issue_scatters(p):
            for j in range(NSUB):
                pltpu.async_copy(buf.at[p, pl.ds(j * SB, SB)],
                                 acc.at[cidx.at[p, j]], ss, add=True)

        def drain(sem):
            for j in range(NSUB):
                pltpu.make_async_copy(g_hbm.at[pl.ds(0, SB)],
                                      buf.at[0, pl.ds(j * SB, SB)],
                                      sem).wait()

        def step(b, p):
            q = 1 - p
            drain(gs)

            @pl.when(b + 2 < NB)
            def _():
                pltpu.async_copy(row_hbm.at[wid, b + 2], ridx.at[p], irs)

            @pl.when(b >= 1)
            def _():
                drain(ss)

            @pl.when(b + 1 < NB)
            def _():
                pltpu.async_copy(col_hbm.at[wid, b + 1], cidx.at[q], ics)
                pltpu.make_async_copy(row_hbm.at[wid, 0], ridx.at[0],
                                      irs).wait()
                issue_gathers(q, b + 1)

            @pl.when(b >= 1)
            def _():
                pltpu.make_async_copy(col_hbm.at[wid, 0], cidx.at[0],
                                      ics).wait()

            issue_scatters(p)

        pltpu.sync_copy(row_hbm.at[wid, 0], ridx.at[0])
        pltpu.sync_copy(col_hbm.at[wid, 0], cidx.at[0])
        issue_gathers(0, 0)
        pltpu.async_copy(row_hbm.at[wid, 1], ridx.at[1], irs)

        @pl.loop(0, NB // 2)
        def _(t):
            step(2 * t, 0)
            step(2 * t + 1, 1)

        drain(ss)
        plsc.subcore_barrier()
        sl = pl.ds(sid * RPS, RPS)

        @pl.when(cid == 0)
        def _():
            pltpu.sync_copy(acc.at[sl], oa_hbm.at[sl])

        @pl.when(cid == 1)
        def _():
            pltpu.sync_copy(acc.at[sl], ob_hbm.at[sl])

    return k(g, rows4, cols4, zeros)


def _sc_degree(rows4, ones, zeros1):
    @functools.partial(
        pl.kernel,
        out_type=[jax.ShapeDtypeStruct((NP, C), _f32)] * 2,
        mesh=_mesh,
        scratch_types=[
            pltpu.VMEM((2, NSUB, SB), jnp.int32),
            pltpu.VMEM((SB,), _f32),
            pltpu.VMEM((RPS,), _f32),
            pltpu.VMEM((RPS, C), _f32),
            pltpu.VMEM_SHARED((NP,), _f32),
            pltpu.SemaphoreType.DMA,
            pltpu.SemaphoreType.DMA,
        ],
        compiler_params=_sc_params,
    )
    def k(row_hbm, ones_hbm, z_hbm, oa_hbm, ob_hbm, ridx, onesb, dbuf,
          rbuf, acc, irs, ss):
        cid = lax.axis_index("c")
        sid = lax.axis_index("s")
        wid = cid * NS + sid
        pltpu.sync_copy(ones_hbm, onesb)
        pltpu.sync_copy(z_hbm.at[pl.ds(sid * RPS, RPS)],
                        acc.at[pl.ds(sid * RPS, RPS)])
        plsc.subcore_barrier()

        def scat(p):
            for j in range(NSUB):
                pltpu.async_copy(onesb, acc.at[ridx.at[p, j]], ss, add=True)

        def drain_ss():
            for j in range(NSUB):
                pltpu.make_async_copy(ones_hbm, onesb, ss).wait()

        pltpu.sync_copy(row_hbm.at[wid, 0], ridx.at[0])

        def stepd(b, p):
            q = 1 - p

            @pl.when(b >= 1)
            def _():
                pltpu.make_async_copy(row_hbm.at[wid, 0], ridx.at[0],
                                      irs).wait()

            scat(p)

            @pl.when(b >= 1)
            def _():
                drain_ss()

            @pl.when(b + 1 < NB)
            def _():
                pltpu.async_copy(row_hbm.at[wid, b + 1], ridx.at[q], irs)

        @pl.loop(0, NB // 2)
        def _(t):
            stepd(2 * t, 0)
            stepd(2 * t + 1, 1)

        drain_ss()
        plsc.subcore_barrier()
        sl = pl.ds(sid * RPS, RPS)
        pltpu.sync_copy(acc.at[sl], dbuf)

        @pl.loop(0, RPS // 16)
        def _(i):
            v = dbuf[pl.ds(i * 16, 16)]
            for kk in range(16):
                rbuf[i * 16 + kk, :] = jnp.broadcast_to(v[kk], (C,))

        @pl.when(cid == 0)
        def _():
            pltpu.sync_copy(rbuf, oa_hbm.at[sl])

        @pl.when(cid == 1)
        def _():
            pltpu.sync_copy(rbuf, ob_hbm.at[sl])

    return k(rows4, ones, zeros1)


def _tc_matmul(x, w):
    BM = 2000

    def body(x_ref, w_ref, o_ref):
        o_ref[...] = jnp.dot(x_ref[...], w_ref[...],
                             preferred_element_type=_f32)

    return pl.pallas_call(
        body,
        grid=(N // BM,),
        in_specs=[
            pl.BlockSpec((BM, F), lambda i: (i, 0)),
            pl.BlockSpec((F, C), lambda i: (0, 0)),
        ],
        out_specs=pl.BlockSpec((BM, C), lambda i: (i, 0)),
        out_shape=jax.ShapeDtypeStruct((NP, C), _f32),
    )(x, w)


_RF = NP * C // 128


def _flat(a):
    return a.reshape(_RF, 128)


def _tc_prep(dega, degb, h0):
    def body(da, db, h, g0_o, di_o, di2_o):
        deg = da[...] + db[...]
        pos = deg > 0.0
        di = jnp.where(pos, lax.rsqrt(deg), 0.0)
        di_o[...] = di
        di2_o[...] = jnp.where(pos, 1.0 / deg, 0.0)
        g0_o[...] = di * h[...]

    return pl.pallas_call(
        body,
        out_shape=[jax.ShapeDtypeStruct((_RF, 128), _f32)] * 3,
    )(_flat(dega), _flat(degb), _flat(h0))


def _tc_combine(sa, sb, scale):
    def body(a, b, s, o):
        o[...] = s[...] * (a[...] + b[...])

    return pl.pallas_call(
        body,
        out_shape=jax.ShapeDtypeStruct((_RF, 128), _f32),
    )(_flat(sa), _flat(sb), scale)


def kernel(x, edge_index, weight):
    row = edge_index[0].astype(jnp.int32)
    col = edge_index[1].astype(jnp.int32)
    padv = jnp.full((EPAD - E,), N, jnp.int32)
    rows4 = jnp.concatenate([row, padv]).reshape(NW, NB, NSUB, SB)
    cols4 = jnp.concatenate([col, padv]).reshape(NW, NB, NSUB, SB)
    zeros = jnp.zeros((NP, C), _f32)
    zeros1 = jnp.zeros((NP,), _f32)
    ones = jnp.ones((SB,), _f32)

    dega, degb = _sc_degree(rows4, ones, zeros1)
    h0 = _tc_matmul(x, weight)
    g0f, dinv, dinv2 = _tc_prep(dega, degb, h0)

    s1a, s1b = _sc_propagate(g0f.reshape(NP, C), rows4, cols4, zeros)
    g1f = _tc_combine(s1a, s1b, dinv2)

    s2a, s2b = _sc_propagate(g1f.reshape(NP, C), rows4, cols4, zeros)
    h2f = _tc_combine(s2a, s2b, dinv)
    return h2f.reshape(NP, C)[:N]


# speedup vs baseline: 119.7269x; 1.0030x over previous
import functools

import jax
import jax.numpy as jnp
from jax import lax
from jax.experimental import pallas as pl
from jax.experimental.pallas import tpu as pltpu
from jax.experimental.pallas import tpu_sc as plsc

N = 100000
F = 128
C = 16
E = 3200000
NC = 2
NS = 16
NW = NC * NS
SB = 128
NSUB = 4
EB = SB * NSUB
NB = 196
EPAD = NW * NB * EB
NP = 100096
RPS = NP // NS

_mesh = plsc.VectorSubcoreMesh(core_axis_name="c", subcore_axis_name="s")
_sc_params = pltpu.CompilerParams(use_tc_tiling_on_sc=False)
_f32 = jnp.float32


def _sc_propagate(g, rows4, cols4, zeros):
    @functools.partial(
        pl.kernel,
        out_type=[jax.ShapeDtypeStruct((NP, C), _f32)] * 2,
        mesh=_mesh,
        scratch_types=[
            pltpu.VMEM((2, EB), jnp.int32),
            pltpu.VMEM((2, EB), jnp.int32),
            pltpu.VMEM((2, EB, C), _f32),
            pltpu.VMEM_SHARED((NP, C), _f32),
            pltpu.SemaphoreType.DMA,
            pltpu.SemaphoreType.DMA,
            pltpu.SemaphoreType.DMA,
            pltpu.SemaphoreType.DMA,
        ],
        compiler_params=_sc_params,
    )
    def k(g_hbm, row_hbm, col_hbm, z_hbm, oa_hbm, ob_hbm, ridx, cidx, buf,
          acc, irs, ics, gs, ss):
        cid = lax.axis_index("c")
        sid = lax.axis_index("s")
        wid = cid * NS + sid
        pltpu.sync_copy(z_hbm.at[pl.ds(sid * RPS, RPS)],
                        acc.at[pl.ds(sid * RPS, RPS)])
        plsc.subcore_barrier()

        def issue_gathers(p, b):
            pltpu.async_copy(g_hbm.at[ridx.at[p]], buf.at[p], gs)

        def issue_scatters(p):
            pltpu.async_copy(buf.at[p], acc.at[cidx.at[p]], ss, add=True)

        def drain(sem):
            pltpu.make_async_copy(g_hbm.at[pl.ds(0, EB)], buf.at[0],
                                  sem).wait()

        def step(b, p):
            q = 1 - p
            drain(gs)

            @pl.when(b + 2 < NB)
            def _():
                pltpu.async_copy(row_hbm.at[wid, b + 2], ridx.at[p], irs)

            @pl.when(b >= 1)
            def _():
                drain(ss)

            @pl.when(b + 1 < NB)
            def _():
                pltpu.async_copy(col_hbm.at[wid, b + 1], cidx.at[q], ics)
                pltpu.make_async_copy(row_hbm.at[wid, 0], ridx.at[0],
                                      irs).wait()
                issue_gathers(q, b + 1)

            @pl.when(b >= 1)
            def _():
                pltpu.make_async_copy(col_hbm.at[wid, 0], cidx.at[0],
                                      ics).wait()

            issue_scatters(p)

        pltpu.sync_copy(row_hbm.at[wid, 0], ridx.at[0])
        pltpu.sync_copy(col_hbm.at[wid, 0], cidx.at[0])
        issue_gathers(0, 0)
        pltpu.async_copy(row_hbm.at[wid, 1], ridx.at[1], irs)

        @pl.loop(0, NB // 2)
        def _(t):
            step(2 * t, 0)
            step(2 * t + 1, 1)

        drain(ss)
        plsc.subcore_barrier()
        sl = pl.ds(sid * RPS, RPS)

        @pl.when(cid == 0)
        def _():
            pltpu.sync_copy(acc.at[sl], oa_hbm.at[sl])

        @pl.when(cid == 1)
        def _():
            pltpu.sync_copy(acc.at[sl], ob_hbm.at[sl])

    return k(g, rows4, cols4, zeros)


def _sc_degree(rows4, ones, zeros1):
    @functools.partial(
        pl.kernel,
        out_type=[jax.ShapeDtypeStruct((NP, C), _f32)] * 2,
        mesh=_mesh,
        scratch_types=[
            pltpu.VMEM((2, EB), jnp.int32),
            pltpu.VMEM((EB,), _f32),
            pltpu.VMEM((RPS,), _f32),
            pltpu.VMEM((RPS, C), _f32),
            pltpu.VMEM_SHARED((NP,), _f32),
            pltpu.SemaphoreType.DMA,
            pltpu.SemaphoreType.DMA,
        ],
        compiler_params=_sc_params,
    )
    def k(row_hbm, ones_hbm, z_hbm, oa_hbm, ob_hbm, ridx, onesb, dbuf,
          rbuf, acc, irs, ss):
        cid = lax.axis_index("c")
        sid = lax.axis_index("s")
        wid = cid * NS + sid
        pltpu.sync_copy(ones_hbm, onesb)
        pltpu.sync_copy(z_hbm.at[pl.ds(sid * RPS, RPS)],
                        acc.at[pl.ds(sid * RPS, RPS)])
        plsc.subcore_barrier()

        def scat(p):
            pltpu.async_copy(onesb, acc.at[ridx.at[p]], ss, add=True)

        def drain_ss():
            pltpu.make_async_copy(ones_hbm, onesb, ss).wait()

        pltpu.sync_copy(row_hbm.at[wid, 0], ridx.at[0])

        def stepd(b, p):
            q = 1 - p

            @pl.when(b >= 1)
            def _():
                pltpu.make_async_copy(row_hbm.at[wid, 0], ridx.at[0],
                                      irs).wait()

            scat(p)

            @pl.when(b >= 1)
            def _():
                drain_ss()

            @pl.when(b + 1 < NB)
            def _():
                pltpu.async_copy(row_hbm.at[wid, b + 1], ridx.at[q], irs)

        @pl.loop(0, NB // 2)
        def _(t):
            stepd(2 * t, 0)
            stepd(2 * t + 1, 1)

        drain_ss()
        plsc.subcore_barrier()
        sl = pl.ds(sid * RPS, RPS)
        pltpu.sync_copy(acc.at[sl], dbuf)

        @pl.loop(0, RPS // 16)
        def _(i):
            v = dbuf[pl.ds(i * 16, 16)]
            for kk in range(16):
                rbuf[i * 16 + kk, :] = jnp.broadcast_to(v[kk], (C,))

        @pl.when(cid == 0)
        def _():
            pltpu.sync_copy(rbuf, oa_hbm.at[sl])

        @pl.when(cid == 1)
        def _():
            pltpu.sync_copy(rbuf, ob_hbm.at[sl])

    return k(rows4, ones, zeros1)


def _tc_matmul(x, w):
    BM = 2000

    def body(x_ref, w_ref, o_ref):
        o_ref[...] = jnp.dot(x_ref[...], w_ref[...],
                             preferred_element_type=_f32)

    return pl.pallas_call(
        body,
        grid=(N // BM,),
        in_specs=[
            pl.BlockSpec((BM, F), lambda i: (i, 0)),
            pl.BlockSpec((F, C), lambda i: (0, 0)),
        ],
        out_specs=pl.BlockSpec((BM, C), lambda i: (i, 0)),
        out_shape=jax.ShapeDtypeStruct((NP, C), _f32),
    )(x, w)


_RF = NP * C // 128


def _flat(a):
    return a.reshape(_RF, 128)


def _tc_prep(dega, degb, h0):
    def body(da, db, h, g0_o, di_o, di2_o):
        deg = da[...] + db[...]
        pos = deg > 0.0
        di = jnp.where(pos, lax.rsqrt(deg), 0.0)
        di_o[...] = di
        di2_o[...] = jnp.where(pos, 1.0 / deg, 0.0)
        g0_o[...] = di * h[...]

    return pl.pallas_call(
        body,
        out_shape=[jax.ShapeDtypeStruct((_RF, 128), _f32)] * 3,
    )(_flat(dega), _flat(degb), _flat(h0))


def _tc_combine(sa, sb, scale):
    def body(a, b, s, o):
        o[...] = s[...] * (a[...] + b[...])

    return pl.pallas_call(
        body,
        out_shape=jax.ShapeDtypeStruct((_RF, 128), _f32),
    )(_flat(sa), _flat(sb), scale)


def kernel(x, edge_index, weight):
    row = edge_index[0].astype(jnp.int32)
    col = edge_index[1].astype(jnp.int32)
    padv = jnp.full((EPAD - E,), N, jnp.int32)
    rows4 = jnp.concatenate([row, padv]).reshape(NW, NB, EB)
    cols4 = jnp.concatenate([col, padv]).reshape(NW, NB, EB)
    zeros = jnp.zeros((NP, C), _f32)
    zeros1 = jnp.zeros((NP,), _f32)
    ones = jnp.ones((EB,), _f32)

    dega, degb = _sc_degree(rows4, ones, zeros1)
    h0 = _tc_matmul(x, weight)
    g0f, dinv, dinv2 = _tc_prep(dega, degb, h0)

    s1a, s1b = _sc_propagate(g0f.reshape(NP, C), rows4, cols4, zeros)
    g1f = _tc_combine(s1a, s1b, dinv2)

    s2a, s2b = _sc_propagate(g1f.reshape(NP, C), rows4, cols4, zeros)
    h2f = _tc_combine(s2a, s2b, dinv)
    return h2f.reshape(NP, C)[:N]
